# Initial kernel scaffold; baseline (speedup 1.0000x reference)
import jax
import jax.numpy as jnp
from jax.experimental import pallas as pl


def kernel(points_xyz, features, w1, b1, w2, b2, w3, b3):
    raise NotImplementedError("write your pallas kernel here")


# speedup vs baseline: 9.7529x; 9.7529x over previous
import functools

import jax
import jax.numpy as jnp
from jax import lax
from jax.experimental import pallas as pl
from jax.experimental.pallas import tpu as pltpu
from jax.experimental.pallas import tpu_sc as plsc

BATCH = 2
N = 16384
C_IN = 128
NPOINT = 1024
RADIUS2 = 0.8 * 0.8
NSAMPLE = 32
D_TAB = 144

NC, NS, L = 2, 16, 16
NW = NC * NS


def _fps_body(xyz_ref, idx_ref, nxz_ref):
    x = xyz_ref[0, 0:128, :]
    y = xyz_ref[0, 128:256, :]
    z = xyz_ref[0, 256:384, :]
    row_i = lax.broadcasted_iota(jnp.int32, (128, 128), 0)
    col_i = lax.broadcasted_iota(jnp.int32, (128, 128), 1)
    idxmat = row_i * 128 + col_i
    lane_i = lax.broadcasted_iota(jnp.int32, (1, 128), 1)
    out_i = (lax.broadcasted_iota(jnp.int32, (8, 128), 0) * 128
             + lax.broadcasted_iota(jnp.int32, (8, 128), 1))

    def body(i, state):
        last, dist, oidx, wx, wy, wz = state
        r = last >> 7
        cl = last & 127
        xr = xyz_ref[0, pl.ds(r, 1), :]
        yr = xyz_ref[0, pl.ds(r + 128, 1), :]
        zr = xyz_ref[0, pl.ds(r + 256, 1), :]
        sel = lane_i == cl
        cx = jnp.sum(jnp.where(sel, xr, 0.0))
        cy = jnp.sum(jnp.where(sel, yr, 0.0))
        cz = jnp.sum(jnp.where(sel, zr, 0.0))
        dx = x - cx
        dy = y - cy
        dz = z - cz
        d = (dx * dx + dy * dy) + dz * dz
        dist = jnp.minimum(dist, d)
        m = jnp.max(dist)
        cand = jnp.where(dist == m, idxmat, jnp.int32(1 << 30))
        nxt = jnp.min(cand)
        oidx = jnp.where(out_i == i, last, oidx)
        wx = jnp.where(out_i == i, cx, wx)
        wy = jnp.where(out_i == i, cy, wy)
        wz = jnp.where(out_i == i, cz, wz)
        return nxt, dist, oidx, wx, wy, wz

    last0 = jnp.int32(0)
    dist0 = jnp.full((128, 128), 1e10, jnp.float32)
    oidx0 = jnp.zeros((8, 128), jnp.int32)
    w0 = jnp.zeros((8, 128), jnp.float32)
    _, _, oidx, wx, wy, wz = lax.fori_loop(
        0, NPOINT, body, (last0, dist0, oidx0, w0, w0, w0))
    idx_ref[0] = oidx
    nxz_ref[0, 0:8, :] = wx
    nxz_ref[0, 8:16, :] = wy
    nxz_ref[0, 16:24, :] = wz


def _fps(xyz_blk):
    return pl.pallas_call(
        _fps_body,
        grid=(BATCH,),
        in_specs=[pl.BlockSpec((1, 384, 128), lambda b: (b, 0, 0))],
        out_specs=[
            pl.BlockSpec((1, 8, 128), lambda b: (b, 0, 0)),
            pl.BlockSpec((1, 24, 128), lambda b: (b, 0, 0)),
        ],
        out_shape=[
            jax.ShapeDtypeStruct((BATCH, 8, 128), jnp.int32),
            jax.ShapeDtypeStruct((BATCH, 24, 128), jnp.float32),
        ],
    )(xyz_blk)


def _ballq_body(xyz_hbm, cen_hbm, out_hbm, x_v, y_v, z_v, cx_v, cy_v, cz_v,
                hits_v, ob_v):
    w = lax.axis_index("s") * NC + lax.axis_index("c")
    b = w % BATCH
    off = w // BATCH
    pltpu.sync_copy(xyz_hbm.at[pl.ds((b * 3 + 0) * N, N)], x_v)
    pltpu.sync_copy(xyz_hbm.at[pl.ds((b * 3 + 1) * N, N)], y_v)
    pltpu.sync_copy(xyz_hbm.at[pl.ds((b * 3 + 2) * N, N)], z_v)
    pltpu.sync_copy(cen_hbm.at[pl.ds((b * 3 + 0) * NPOINT, NPOINT)], cx_v)
    pltpu.sync_copy(cen_hbm.at[pl.ds((b * 3 + 1) * NPOINT, NPOINT)], cy_v)
    pltpu.sync_copy(cen_hbm.at[pl.ds((b * 3 + 2) * NPOINT, NPOINT)], cz_v)

    io16 = lax.iota(jnp.int32, L)
    rr = jnp.float32(RADIUS2)

    def per_centroid(g, _):
        cloc = off + 16 * g
        cvec = jnp.zeros((L,), jnp.int32) + cloc
        cx = plsc.load_gather(cx_v, [cvec])
        cy = plsc.load_gather(cy_v, [cvec])
        cz = plsc.load_gather(cz_v, [cvec])

        def cond(st):
            ci, cnt, fmin = st
            return (cnt < NSAMPLE) & (ci < N // L)

        def step(st):
            ci, cnt, fmin = st
            base = ci * L
            xv = x_v[pl.ds(base, L)]
            yv = y_v[pl.ds(base, L)]
            zv = z_v[pl.ds(base, L)]
            dx = xv - cx
            dy = yv - cy
            dz = zv - cz
            d2 = (dx * dx + dy * dy) + dz * dz
            msk = d2 < rr
            idxv = base + io16
            plsc.store_compressed(hits_v.at[pl.ds(cnt, L)], idxv, mask=msk)
            cnt = cnt + jnp.sum(msk.astype(jnp.int32))
            fmin = jnp.minimum(fmin, jnp.min(jnp.where(msk, idxv, N)))
            return ci + 1, cnt, fmin

        _, cnt, fmin = lax.while_loop(
            cond, step, (jnp.int32(0), jnp.int32(0), jnp.int32(N)))
        h0 = hits_v[pl.ds(0, L)]
        h1 = hits_v[pl.ds(L, L)]
        first = jnp.zeros((L,), jnp.int32) + fmin
        cntv = jnp.zeros((L,), jnp.int32) + cnt
        ob_v[g, pl.ds(0, L)] = jnp.where(io16 < cntv, h0, first)
        ob_v[g, pl.ds(L, L)] = jnp.where(io16 + L < cntv, h1, first)
        return 0

    lax.fori_loop(0, NPOINT // 16, per_centroid, 0)
    pltpu.sync_copy(ob_v, out_hbm.at[b, off])


def _ball_query(xyz_flat, cen_flat):
    mesh = plsc.VectorSubcoreMesh(core_axis_name="c", subcore_axis_name="s",
                                  num_cores=NC, num_subcores=NS)
    f = functools.partial(
        pl.kernel,
        mesh=mesh,
        compiler_params=pltpu.CompilerParams(needs_layout_passes=False),
        out_type=jax.ShapeDtypeStruct((BATCH, 16, NPOINT // 16, NSAMPLE),
                                      jnp.int32),
        scratch_types=[
            pltpu.VMEM((N,), jnp.float32),
            pltpu.VMEM((N,), jnp.float32),
            pltpu.VMEM((N,), jnp.float32),
            pltpu.VMEM((NPOINT,), jnp.float32),
            pltpu.VMEM((NPOINT,), jnp.float32),
            pltpu.VMEM((NPOINT,), jnp.float32),
            pltpu.VMEM((64,), jnp.int32),
            pltpu.VMEM((64, NSAMPLE), jnp.int32),
        ],
    )(_ballq_body)
    return f(xyz_flat, cen_flat)


_GCHUNK = 128


def _gather_body(tab_hbm, idx_hbm, out_hbm, idx_v, rows_v, sem):
    w = lax.axis_index("s") * NC + lax.axis_index("c")
    b = w % BATCH
    base = (w // BATCH) * (BATCH * NPOINT * NSAMPLE // NW)
    boff = b * N
    nchunk = BATCH * NPOINT * NSAMPLE // NW // _GCHUNK
    for j in range(nchunk):
        r0 = base + j * _GCHUNK
        pltpu.sync_copy(
            idx_hbm.at[pl.ds(b * NPOINT * NSAMPLE + r0, _GCHUNK)], idx_v)
        for t in range(_GCHUNK // L):
            v = idx_v[pl.ds(t * L, L)] + boff
            idx_v[pl.ds(t * L, L)] = jnp.minimum(
                jnp.maximum(v, 0), BATCH * N - 1)
        pltpu.async_copy(tab_hbm.at[idx_v], rows_v, sem).wait()
        pltpu.sync_copy(rows_v, out_hbm.at[b, pl.ds(r0, _GCHUNK)])


def _gather(tab, gidx):
    mesh = plsc.VectorSubcoreMesh(core_axis_name="c", subcore_axis_name="s",
                                  num_cores=NC, num_subcores=NS)
    f = functools.partial(
        pl.kernel,
        mesh=mesh,
        compiler_params=pltpu.CompilerParams(use_tc_tiling_on_sc=False),
        out_type=jax.ShapeDtypeStruct((BATCH, NPOINT * NSAMPLE, D_TAB),
                                      jnp.float32),
        scratch_types=[
            pltpu.VMEM((_GCHUNK,), jnp.int32),
            pltpu.VMEM((_GCHUNK, D_TAB), jnp.float32),
            pltpu.SemaphoreType.DMA,
        ],
    )(_gather_body)
    return f(tab, gidx)


_ROWS = 4096


def _mlp_body(g_ref, c_ref, w1_ref, b1_ref, w2_ref, b2_ref, w3_ref, b3_ref,
              out_ref):
    g = g_ref[0]
    cen = c_ref[0]
    crep = jnp.broadcast_to(cen[:, None, :], (128, 32, D_TAB))
    crep = crep.reshape(_ROWS, D_TAB)
    x = g - crep

    def dense(v, w_ref, b_ref):
        y = lax.dot_general(v, w_ref[...], (((1,), (0,)), ((), ())),
                            precision=lax.Precision.HIGHEST,
                            preferred_element_type=jnp.float32)
        return jnp.maximum(y + b_ref[0][None, :], 0.0)

    h = dense(x, w1_ref, b1_ref)
    h = dense(h, w2_ref, b2_ref)
    h = dense(h, w3_ref, b3_ref)
    hm = jnp.max(h.reshape(128, 32, 512), axis=1)
    out_ref[0] = hm.T


def _mlp(grouped, centers_p, w1p, b1, w2t, b2, w3t, b3):
    nblk = NPOINT * NSAMPLE // _ROWS
    return pl.pallas_call(
        _mlp_body,
        grid=(BATCH, nblk),
        in_specs=[
            pl.BlockSpec((1, _ROWS, D_TAB), lambda b, j: (b, j, 0)),
            pl.BlockSpec((1, 128, D_TAB), lambda b, j: (b, j, 0)),
            pl.BlockSpec((D_TAB, 128), lambda b, j: (0, 0)),
            pl.BlockSpec((1, 128), lambda b, j: (0, 0)),
            pl.BlockSpec((128, 256), lambda b, j: (0, 0)),
            pl.BlockSpec((1, 256), lambda b, j: (0, 0)),
            pl.BlockSpec((256, 512), lambda b, j: (0, 0)),
            pl.BlockSpec((1, 512), lambda b, j: (0, 0)),
        ],
        out_specs=pl.BlockSpec((1, 512, 128), lambda b, j: (b, 0, j)),
        out_shape=jax.ShapeDtypeStruct((BATCH, 512, NPOINT), jnp.float32),
    )(grouped, centers_p, w1p, b1, w2t, b2, w3t, b3)


def kernel(points_xyz, features, w1, b1, w2, b2, w3, b3):
    xyz_t = jnp.transpose(points_xyz, (0, 2, 1))
    xyz_blk = xyz_t.reshape(BATCH, 3 * 128, 128)

    idx_blk, nxz_blk = _fps(xyz_blk)
    samp_idx = idx_blk.reshape(BATCH, NPOINT)
    cen3 = nxz_blk.reshape(BATCH, 3, NPOINT)
    new_xyz = jnp.transpose(cen3, (0, 2, 1))

    gidx4 = _ball_query(xyz_t.reshape(-1), cen3.reshape(-1))
    gidx = jnp.transpose(gidx4, (0, 2, 1, 3)).reshape(-1)

    tab = jnp.concatenate(
        [points_xyz,
         jnp.zeros((BATCH, N, D_TAB - 3 - C_IN), jnp.float32),
         jnp.transpose(features, (0, 2, 1))], axis=-1)
    grouped = _gather(tab.reshape(BATCH * N, D_TAB), gidx)

    centers_p = jnp.concatenate(
        [new_xyz, jnp.zeros((BATCH, NPOINT, D_TAB - 3), jnp.float32)],
        axis=-1)

    w1p = jnp.zeros((D_TAB, 128), jnp.float32)
    w1p = w1p.at[0:3].set(w1[:, 0:3].T)
    w1p = w1p.at[16:144].set(w1[:, 3:131].T)
    pooled = _mlp(grouped, centers_p, w1p, b1[None, :], w2.T, b2[None, :],
                  w3.T, b3[None, :])

    return new_xyz, pooled, samp_idx


# speedup vs baseline: 15.0126x; 1.5393x over previous
import functools

import jax
import jax.numpy as jnp
from jax import lax
from jax.experimental import pallas as pl
from jax.experimental.pallas import tpu as pltpu
from jax.experimental.pallas import tpu_sc as plsc

BATCH = 2
N = 16384
C_IN = 128
NPOINT = 1024
RADIUS2 = 0.8 * 0.8
NSAMPLE = 32
D_TAB = 144

NC, NS, L = 2, 16, 16
NW = NC * NS


def _fps_body(xyz_ref, idx_ref, nxz_ref):
    xs = [xyz_ref[b, 0:128, :] for b in range(BATCH)]
    ys = [xyz_ref[b, 128:256, :] for b in range(BATCH)]
    zs = [xyz_ref[b, 256:384, :] for b in range(BATCH)]
    row_i = lax.broadcasted_iota(jnp.int32, (128, 128), 0)
    col_i = lax.broadcasted_iota(jnp.int32, (128, 128), 1)
    idxmat = row_i * 128 + col_i
    lane_i = lax.broadcasted_iota(jnp.int32, (1, 128), 1)
    out_i = (lax.broadcasted_iota(jnp.int32, (8, 128), 0) * 128
             + lax.broadcasted_iota(jnp.int32, (8, 128), 1))

    def body(i, state):
        last, dist, oidx, wx, wy, wz = state
        out = [[], [], [], [], [], []]
        for b in range(BATCH):
            r = last[b] >> 7
            cl = last[b] & 127
            xr = xyz_ref[b, pl.ds(r, 1), :]
            yr = xyz_ref[b, pl.ds(r + 128, 1), :]
            zr = xyz_ref[b, pl.ds(r + 256, 1), :]
            sel = lane_i == cl
            cx = jnp.sum(jnp.where(sel, xr, 0.0))
            cy = jnp.sum(jnp.where(sel, yr, 0.0))
            cz = jnp.sum(jnp.where(sel, zr, 0.0))
            dx = xs[b] - cx
            dy = ys[b] - cy
            dz = zs[b] - cz
            d = (dx * dx + dy * dy) + dz * dz
            db = jnp.minimum(dist[b], d)
            m = jnp.max(db)
            cand = jnp.where(db == m, idxmat, jnp.int32(1 << 30))
            nxt = jnp.min(cand)
            out[0].append(nxt)
            out[1].append(db)
            out[2].append(jnp.where(out_i == i, last[b], oidx[b]))
            out[3].append(jnp.where(out_i == i, cx, wx[b]))
            out[4].append(jnp.where(out_i == i, cy, wy[b]))
            out[5].append(jnp.where(out_i == i, cz, wz[b]))
        return tuple(tuple(o) for o in out)

    last0 = (jnp.int32(0),) * BATCH
    dist0 = (jnp.full((128, 128), 1e10, jnp.float32),) * BATCH
    oidx0 = (jnp.zeros((8, 128), jnp.int32),) * BATCH
    w0 = (jnp.zeros((8, 128), jnp.float32),) * BATCH
    _, _, oidx, wx, wy, wz = lax.fori_loop(
        0, NPOINT, body, (last0, dist0, oidx0, w0, w0, w0))
    for b in range(BATCH):
        idx_ref[b] = oidx[b]
        nxz_ref[b, 0:8, :] = wx[b]
        nxz_ref[b, 8:16, :] = wy[b]
        nxz_ref[b, 16:24, :] = wz[b]


def _fps(xyz_blk):
    return pl.pallas_call(
        _fps_body,
        out_shape=[
            jax.ShapeDtypeStruct((BATCH, 8, 128), jnp.int32),
            jax.ShapeDtypeStruct((BATCH, 24, 128), jnp.float32),
        ],
    )(xyz_blk)


def _ballq_body(xyz_hbm, cen_hbm, out_hbm, x_v, y_v, z_v, cx_v, cy_v, cz_v,
                hits_v, ob_v):
    w = lax.axis_index("s") * NC + lax.axis_index("c")
    b = w % BATCH
    off = w // BATCH
    pltpu.sync_copy(xyz_hbm.at[pl.ds((b * 3 + 0) * N, N)], x_v)
    pltpu.sync_copy(xyz_hbm.at[pl.ds((b * 3 + 1) * N, N)], y_v)
    pltpu.sync_copy(xyz_hbm.at[pl.ds((b * 3 + 2) * N, N)], z_v)
    pltpu.sync_copy(cen_hbm.at[pl.ds((b * 3 + 0) * NPOINT, NPOINT)], cx_v)
    pltpu.sync_copy(cen_hbm.at[pl.ds((b * 3 + 1) * NPOINT, NPOINT)], cy_v)
    pltpu.sync_copy(cen_hbm.at[pl.ds((b * 3 + 2) * NPOINT, NPOINT)], cz_v)

    io16 = lax.iota(jnp.int32, L)
    rr = jnp.float32(RADIUS2)

    def per_centroid(g, _):
        cloc = off + 16 * g
        cvec = jnp.zeros((L,), jnp.int32) + cloc
        cx = plsc.load_gather(cx_v, [cvec])
        cy = plsc.load_gather(cy_v, [cvec])
        cz = plsc.load_gather(cz_v, [cvec])

        def cond(st):
            ci, cnt, fmin_v = st
            return (cnt < NSAMPLE) & (ci < N // (4 * L))

        def step(st):
            ci, cnt, fmin_v = st
            base = ci * (4 * L)
            msks, pcs = [], []
            for j in range(4):
                bj = base + j * L
                xv = x_v[pl.ds(bj, L)]
                yv = y_v[pl.ds(bj, L)]
                zv = z_v[pl.ds(bj, L)]
                dx = xv - cx
                dy = yv - cy
                dz = zv - cz
                d2 = (dx * dx + dy * dy) + dz * dz
                msk = d2 < rr
                msks.append(msk)
                pc = plsc.all_reduce_population_count(msk)
                pcs.append(pc)
                ffs = plsc.all_reduce_ffs(msk)
                hit1 = jnp.where(pc > 0, bj + ffs, N)
                fmin_v = jnp.minimum(fmin_v, hit1)
            c0 = cnt
            c1 = c0 + jnp.max(pcs[0])
            c2 = c1 + jnp.max(pcs[1])
            c3 = c2 + jnp.max(pcs[2])
            c4 = c3 + jnp.max(pcs[3])
            plsc.store_compressed(hits_v.at[pl.ds(c0, L)], base + io16,
                                  mask=msks[0])
            plsc.store_compressed(hits_v.at[pl.ds(c1, L)], base + L + io16,
                                  mask=msks[1])
            plsc.store_compressed(hits_v.at[pl.ds(c2, L)], base + 2 * L + io16,
                                  mask=msks[2])
            plsc.store_compressed(hits_v.at[pl.ds(c3, L)], base + 3 * L + io16,
                                  mask=msks[3])
            return ci + 1, c4, fmin_v

        _, cnt, fmin_v = lax.while_loop(
            cond, step,
            (jnp.int32(0), jnp.int32(0), jnp.full((L,), N, jnp.int32)))
        h0 = hits_v[pl.ds(0, L)]
        h1 = hits_v[pl.ds(L, L)]
        first = fmin_v
        cntv = jnp.zeros((L,), jnp.int32) + cnt
        ob_v[g, pl.ds(0, L)] = jnp.where(io16 < cntv, h0, first)
        ob_v[g, pl.ds(L, L)] = jnp.where(io16 + L < cntv, h1, first)
        return 0

    lax.fori_loop(0, NPOINT // 16, per_centroid, 0)
    pltpu.sync_copy(ob_v, out_hbm.at[b, off])


def _ball_query(xyz_flat, cen_flat):
    mesh = plsc.VectorSubcoreMesh(core_axis_name="c", subcore_axis_name="s",
                                  num_cores=NC, num_subcores=NS)
    f = functools.partial(
        pl.kernel,
        mesh=mesh,
        compiler_params=pltpu.CompilerParams(needs_layout_passes=False),
        out_type=jax.ShapeDtypeStruct((BATCH, 16, NPOINT // 16, NSAMPLE),
                                      jnp.int32),
        scratch_types=[
            pltpu.VMEM((N,), jnp.float32),
            pltpu.VMEM((N,), jnp.float32),
            pltpu.VMEM((N,), jnp.float32),
            pltpu.VMEM((NPOINT,), jnp.float32),
            pltpu.VMEM((NPOINT,), jnp.float32),
            pltpu.VMEM((NPOINT,), jnp.float32),
            pltpu.VMEM((128,), jnp.int32),
            pltpu.VMEM((64, NSAMPLE), jnp.int32),
        ],
    )(_ballq_body)
    return f(xyz_flat, cen_flat)


_GCHUNK = 128


def _gather_body(tab_hbm, idx_hbm, out_hbm, idx_v, rows_v, sem):
    w = lax.axis_index("s") * NC + lax.axis_index("c")
    b = w % BATCH
    base = (w // BATCH) * (BATCH * NPOINT * NSAMPLE // NW)
    boff = b * N
    nchunk = BATCH * NPOINT * NSAMPLE // NW // _GCHUNK
    for j in range(nchunk):
        r0 = base + j * _GCHUNK
        pltpu.sync_copy(
            idx_hbm.at[pl.ds(b * NPOINT * NSAMPLE + r0, _GCHUNK)], idx_v)
        for t in range(_GCHUNK // L):
            v = idx_v[pl.ds(t * L, L)] + boff
            idx_v[pl.ds(t * L, L)] = jnp.minimum(
                jnp.maximum(v, 0), BATCH * N - 1)
        pltpu.async_copy(tab_hbm.at[idx_v], rows_v, sem).wait()
        pltpu.sync_copy(rows_v, out_hbm.at[b, pl.ds(r0, _GCHUNK)])


def _gather(tab, gidx):
    mesh = plsc.VectorSubcoreMesh(core_axis_name="c", subcore_axis_name="s",
                                  num_cores=NC, num_subcores=NS)
    f = functools.partial(
        pl.kernel,
        mesh=mesh,
        compiler_params=pltpu.CompilerParams(use_tc_tiling_on_sc=False),
        out_type=jax.ShapeDtypeStruct((BATCH, NPOINT * NSAMPLE, D_TAB),
                                      jnp.float32),
        scratch_types=[
            pltpu.VMEM((_GCHUNK,), jnp.int32),
            pltpu.VMEM((_GCHUNK, D_TAB), jnp.float32),
            pltpu.SemaphoreType.DMA,
        ],
    )(_gather_body)
    return f(tab, gidx)


_ROWS = 4096


def _mlp_body(g_ref, c_ref, w1_ref, b1_ref, w2_ref, b2_ref, w3_ref, b3_ref,
              out_ref):
    g = g_ref[0]
    cen = c_ref[0]
    crep = jnp.broadcast_to(cen[:, None, :], (128, 32, D_TAB))
    crep = crep.reshape(_ROWS, D_TAB)
    x = g - crep

    def dense(v, w_ref, b_ref):
        y = lax.dot_general(v, w_ref[...], (((1,), (0,)), ((), ())),
                            precision=lax.Precision.HIGHEST,
                            preferred_element_type=jnp.float32)
        return jnp.maximum(y + b_ref[0][None, :], 0.0)

    h = dense(x, w1_ref, b1_ref)
    h = dense(h, w2_ref, b2_ref)
    h = dense(h, w3_ref, b3_ref)
    hm = jnp.max(h.reshape(128, 32, 512), axis=1)
    out_ref[0] = hm.T


def _mlp(grouped, centers_p, w1p, b1, w2t, b2, w3t, b3):
    nblk = NPOINT * NSAMPLE // _ROWS
    return pl.pallas_call(
        _mlp_body,
        grid=(BATCH, nblk),
        in_specs=[
            pl.BlockSpec((1, _ROWS, D_TAB), lambda b, j: (b, j, 0)),
            pl.BlockSpec((1, 128, D_TAB), lambda b, j: (b, j, 0)),
            pl.BlockSpec((D_TAB, 128), lambda b, j: (0, 0)),
            pl.BlockSpec((1, 128), lambda b, j: (0, 0)),
            pl.BlockSpec((128, 256), lambda b, j: (0, 0)),
            pl.BlockSpec((1, 256), lambda b, j: (0, 0)),
            pl.BlockSpec((256, 512), lambda b, j: (0, 0)),
            pl.BlockSpec((1, 512), lambda b, j: (0, 0)),
        ],
        out_specs=pl.BlockSpec((1, 512, 128), lambda b, j: (b, 0, j)),
        out_shape=jax.ShapeDtypeStruct((BATCH, 512, NPOINT), jnp.float32),
    )(grouped, centers_p, w1p, b1, w2t, b2, w3t, b3)


def kernel(points_xyz, features, w1, b1, w2, b2, w3, b3):
    xyz_t = jnp.transpose(points_xyz, (0, 2, 1))
    xyz_blk = xyz_t.reshape(BATCH, 3 * 128, 128)

    idx_blk, nxz_blk = _fps(xyz_blk)
    samp_idx = idx_blk.reshape(BATCH, NPOINT)
    cen3 = nxz_blk.reshape(BATCH, 3, NPOINT)
    new_xyz = jnp.transpose(cen3, (0, 2, 1))

    gidx4 = _ball_query(xyz_t.reshape(-1), cen3.reshape(-1))
    gidx = jnp.transpose(gidx4, (0, 2, 1, 3)).reshape(-1)

    tab = jnp.concatenate(
        [points_xyz,
         jnp.zeros((BATCH, N, D_TAB - 3 - C_IN), jnp.float32),
         jnp.transpose(features, (0, 2, 1))], axis=-1)
    grouped = _gather(tab.reshape(BATCH * N, D_TAB), gidx)

    centers_p = jnp.concatenate(
        [new_xyz, jnp.zeros((BATCH, NPOINT, D_TAB - 3), jnp.float32)],
        axis=-1)

    w1p = jnp.zeros((D_TAB, 128), jnp.float32)
    w1p = w1p.at[0:3].set(w1[:, 0:3].T)
    w1p = w1p.at[16:144].set(w1[:, 3:131].T)
    pooled = _mlp(grouped, centers_p, w1p, b1[None, :], w2.T, b2[None, :],
                  w3.T, b3[None, :])

    return new_xyz, pooled, samp_idx
